# baseline (speedup 1.0000x reference)
import jax
import jax.numpy as jnp
from jax import lax
from jax.experimental import pallas as pl
from jax.experimental.pallas import tpu as pltpu
from jax.experimental.pallas import tpu_sc as plsc

EMBED = 64
ENT = 1000000
B = 16384
NC, NS, L = 2, 16, 16
NW = NC * NS
ROWS = B // NW
CR = 128
NCH = ROWS // CR


def _body(idx_hbm, paths_hbm, ents_hbm,
          pos_out_hbm, neg_out_hbm,
          pidx_v, aidx_v, bidx_v, ppair_v, apair_v, bpair_v, parv,
          pb0, pb1, ab0, ab1, bb0, bb1, pos_s, neg_s, sem):
    wid = lax.axis_index("s") * NC + lax.axis_index("c")
    base = wid * ROWS

    pltpu.sync_copy(idx_hbm.at[pl.ds(base, ROWS)], pidx_v)
    pltpu.sync_copy(idx_hbm.at[pl.ds(B + base, ROWS)], aidx_v)
    pltpu.sync_copy(idx_hbm.at[pl.ds(2 * B + base, ROWS)], bidx_v)

    for t in range(ROWS // L):
        sl = pl.ds(t * L, L)
        pv = pidx_v[sl]
        av = aidx_v[sl]
        bv = bidx_v[sl]
        ppair_v[sl] = pv >> 1
        apair_v[sl] = av >> 1
        bpair_v[sl] = bv >> 1
        parv[sl] = (pv & 1) | ((av & 1) << 1) | ((bv & 1) << 2)

    pbufs, abufs, bbufs = (pb0, pb1), (ab0, ab1), (bb0, bb1)

    def fire(c):
        p = c & 1
        sl = pl.ds(c * CR, CR)
        return [pltpu.async_copy(paths_hbm.at[ppair_v.at[sl]], pbufs[p], sem),
                pltpu.async_copy(ents_hbm.at[apair_v.at[sl]], abufs[p], sem),
                pltpu.async_copy(ents_hbm.at[bpair_v.at[sl]], bbufs[p], sem)]

    iota = lax.iota(jnp.int32, L)
    perms = [iota ^ s for s in (8, 4, 2, 1)]

    def fold(v):
        for p in perms:
            v = v + v.at[p].get(mode="promise_in_bounds")
        return v

    def compute(c):
        p = c & 1
        pb, ab, bb = pbufs[p], abufs[p], bbufs[p]

        def group(g, carry):
            posvec = jnp.zeros((L,), jnp.float32)
            negvec = jnp.zeros((L,), jnp.float32)
            pvec = parv[pl.ds(c * CR + g * L, L)]
            for rr in range(L):
                i = g * L + rr
                bits = pvec.at[jnp.full((L,), rr, jnp.int32)].get(
                    mode="promise_in_bounds")
                mp = (bits & 1).astype(jnp.float32)
                ma = ((bits >> 1) & 1).astype(jnp.float32)
                mb = ((bits >> 2) & 1).astype(jnp.float32)
                ap = jnp.zeros((L,), jnp.float32)
                an = jnp.zeros((L,), jnp.float32)
                for k in range(EMBED // L):
                    lo = pl.ds(k * L, L)
                    hi = pl.ds(EMBED + k * L, L)
                    plo = pb[i, lo]
                    alo = ab[i, lo]
                    blo = bb[i, lo]
                    pvv = plo + mp * (pb[i, hi] - plo)
                    av = alo + ma * (ab[i, hi] - alo)
                    bv = blo + mb * (bb[i, hi] - blo)
                    ap = ap + pvv * av
                    an = an + pvv * bv
                lane = iota == rr
                posvec = jnp.where(lane, fold(ap), posvec)
                negvec = jnp.where(lane, fold(an), negvec)
            out_sl = pl.ds(c * CR + g * L, L)
            pos_s[out_sl] = posvec
            neg_s[out_sl] = negvec
            return carry

        lax.fori_loop(0, CR // L, group, 0)

    handles = fire(0)
    for c in range(NCH):
        if c + 1 < NCH:
            nxt = fire(c + 1)
        for h in handles:
            h.wait()
        compute(c)
        if c + 1 < NCH:
            handles = nxt

    pltpu.sync_copy(pos_s, pos_out_hbm.at[pl.ds(base, ROWS)])
    pltpu.sync_copy(neg_s, neg_out_hbm.at[pl.ds(base, ROWS)])


def kernel(ents_path_idxs, embeddings_entities, embeddings_paths):
    idx = ents_path_idxs.astype(jnp.int32).T.reshape(3 * B)
    ents2 = embeddings_entities.reshape(ENT // 2, 2 * EMBED)
    paths2 = embeddings_paths.reshape(ENT // 2, 2 * EMBED)

    mesh = plsc.VectorSubcoreMesh(core_axis_name="c", subcore_axis_name="s",
                                  num_cores=NC, num_subcores=NS)
    run = pl.kernel(
        _body,
        out_type=[jax.ShapeDtypeStruct((B,), jnp.float32),
                  jax.ShapeDtypeStruct((B,), jnp.float32)],
        mesh=mesh,
        compiler_params=pltpu.CompilerParams(use_tc_tiling_on_sc=True),
        scratch_types=[
            pltpu.VMEM((ROWS,), jnp.int32),
            pltpu.VMEM((ROWS,), jnp.int32),
            pltpu.VMEM((ROWS,), jnp.int32),
            pltpu.VMEM((ROWS,), jnp.int32),
            pltpu.VMEM((ROWS,), jnp.int32),
            pltpu.VMEM((ROWS,), jnp.int32),
            pltpu.VMEM((ROWS,), jnp.int32),
            pltpu.VMEM((CR, 2 * EMBED), jnp.float32),
            pltpu.VMEM((CR, 2 * EMBED), jnp.float32),
            pltpu.VMEM((CR, 2 * EMBED), jnp.float32),
            pltpu.VMEM((CR, 2 * EMBED), jnp.float32),
            pltpu.VMEM((CR, 2 * EMBED), jnp.float32),
            pltpu.VMEM((CR, 2 * EMBED), jnp.float32),
            pltpu.VMEM((ROWS,), jnp.float32),
            pltpu.VMEM((ROWS,), jnp.float32),
            pltpu.SemaphoreType.DMA,
        ],
    )
    pos, neg = run(idx, paths2, ents2)

    d1 = jnp.take(embeddings_entities, ents_path_idxs[:, 1], axis=0).sum()
    d2 = jnp.take(embeddings_paths, ents_path_idxs[:, 0], axis=0).sum()
    z = (d1 - d1) + (d2 - d2)
    return pos.reshape(B, 1) + z, neg.reshape(B, 1) + z


# speedup vs baseline: 1.1167x; 1.1167x over previous
import jax
import jax.numpy as jnp
from jax import lax
from jax.experimental import pallas as pl
from jax.experimental.pallas import tpu as pltpu
from jax.experimental.pallas import tpu_sc as plsc

EMBED = 64
ENT = 1000000
B = 16384
NC, NS, L = 2, 16, 16
NW = NC * NS
ROWS = B // NW
CR = 128
NCH = ROWS // CR


def _body(idx_hbm, paths_hbm, ents_hbm,
          pos_out_hbm, neg_out_hbm,
          pidx_v, aidx_v, bidx_v, ppair_v, apair_v, bpair_v, parv,
          pb0, pb1, ab0, ab1, bb0, bb1, pos_s, neg_s, sem):
    wid = lax.axis_index("s") * NC + lax.axis_index("c")
    base = wid * ROWS

    pltpu.sync_copy(idx_hbm.at[pl.ds(base, ROWS)], pidx_v)
    pltpu.sync_copy(idx_hbm.at[pl.ds(B + base, ROWS)], aidx_v)
    pltpu.sync_copy(idx_hbm.at[pl.ds(2 * B + base, ROWS)], bidx_v)

    for t in range(ROWS // L):
        sl = pl.ds(t * L, L)
        pv = pidx_v[sl]
        av = aidx_v[sl]
        bv = bidx_v[sl]
        ppair_v[sl] = pv >> 1
        apair_v[sl] = av >> 1
        bpair_v[sl] = bv >> 1
        parv[sl] = (pv & 1) | ((av & 1) << 1) | ((bv & 1) << 2)

    pbufs, abufs, bbufs = (pb0, pb1), (ab0, ab1), (bb0, bb1)

    def fire(c):
        p = c & 1
        sl = pl.ds(c * CR, CR)
        return [pltpu.async_copy(paths_hbm.at[ppair_v.at[sl]], pbufs[p], sem),
                pltpu.async_copy(ents_hbm.at[apair_v.at[sl]], abufs[p], sem),
                pltpu.async_copy(ents_hbm.at[bpair_v.at[sl]], bbufs[p], sem)]

    iota = lax.iota(jnp.int32, L)
    perms = [iota ^ s for s in (8, 4, 2, 1)]

    def fold(v):
        for p in perms:
            v = v + v.at[p].get(mode="promise_in_bounds")
        return v

    def compute(c):
        p = c & 1
        pb, ab, bb = pbufs[p], abufs[p], bbufs[p]

        def group(g, carry):
            posvec = jnp.zeros((L,), jnp.float32)
            negvec = jnp.zeros((L,), jnp.float32)
            pvec = parv[pl.ds(c * CR + g * L, L)]
            for rr in range(L):
                i = g * L + rr
                bits = pvec.at[jnp.full((L,), rr, jnp.int32)].get(
                    mode="promise_in_bounds")
                mp = (bits & 1).astype(jnp.float32)
                ma = ((bits >> 1) & 1).astype(jnp.float32)
                mb = ((bits >> 2) & 1).astype(jnp.float32)
                ap = jnp.zeros((L,), jnp.float32)
                an = jnp.zeros((L,), jnp.float32)
                for k in range(EMBED // L):
                    lo = pl.ds(k * L, L)
                    hi = pl.ds(EMBED + k * L, L)
                    plo = pb[i, lo]
                    alo = ab[i, lo]
                    blo = bb[i, lo]
                    pvv = plo + mp * (pb[i, hi] - plo)
                    av = alo + ma * (ab[i, hi] - alo)
                    bv = blo + mb * (bb[i, hi] - blo)
                    ap = ap + pvv * av
                    an = an + pvv * bv
                lane = iota == rr
                posvec = jnp.where(lane, fold(ap), posvec)
                negvec = jnp.where(lane, fold(an), negvec)
            out_sl = pl.ds(c * CR + g * L, L)
            pos_s[out_sl] = posvec
            neg_s[out_sl] = negvec
            return carry

        lax.fori_loop(0, CR // L, group, 0)

    handles = fire(0)
    for c in range(NCH):
        if c + 1 < NCH:
            nxt = fire(c + 1)
        for h in handles:
            h.wait()
        compute(c)
        if c + 1 < NCH:
            handles = nxt

    pltpu.sync_copy(pos_s, pos_out_hbm.at[pl.ds(base, ROWS)])
    pltpu.sync_copy(neg_s, neg_out_hbm.at[pl.ds(base, ROWS)])


def kernel(ents_path_idxs, embeddings_entities, embeddings_paths):
    idx = ents_path_idxs.astype(jnp.int32).T.reshape(3 * B)
    ents2 = embeddings_entities.reshape(ENT // 2, 2 * EMBED)
    paths2 = embeddings_paths.reshape(ENT // 2, 2 * EMBED)

    mesh = plsc.VectorSubcoreMesh(core_axis_name="c", subcore_axis_name="s",
                                  num_cores=NC, num_subcores=NS)
    run = pl.kernel(
        _body,
        out_type=[jax.ShapeDtypeStruct((B,), jnp.float32),
                  jax.ShapeDtypeStruct((B,), jnp.float32)],
        mesh=mesh,
        compiler_params=pltpu.CompilerParams(use_tc_tiling_on_sc=True),
        scratch_types=[
            pltpu.VMEM((ROWS,), jnp.int32),
            pltpu.VMEM((ROWS,), jnp.int32),
            pltpu.VMEM((ROWS,), jnp.int32),
            pltpu.VMEM((ROWS,), jnp.int32),
            pltpu.VMEM((ROWS,), jnp.int32),
            pltpu.VMEM((ROWS,), jnp.int32),
            pltpu.VMEM((ROWS,), jnp.int32),
            pltpu.VMEM((CR, 2 * EMBED), jnp.float32),
            pltpu.VMEM((CR, 2 * EMBED), jnp.float32),
            pltpu.VMEM((CR, 2 * EMBED), jnp.float32),
            pltpu.VMEM((CR, 2 * EMBED), jnp.float32),
            pltpu.VMEM((CR, 2 * EMBED), jnp.float32),
            pltpu.VMEM((CR, 2 * EMBED), jnp.float32),
            pltpu.VMEM((ROWS,), jnp.float32),
            pltpu.VMEM((ROWS,), jnp.float32),
            pltpu.SemaphoreType.DMA,
        ],
    )
    pos, neg = run(idx, paths2, ents2)
    return pos.reshape(B, 1), neg.reshape(B, 1)
